# baseline (speedup 1.0000x reference)
import functools

import jax
import jax.numpy as jnp
from jax import lax
from jax.experimental import pallas as pl
from jax.experimental.pallas import tpu as pltpu
from jax.experimental.pallas import tpu_sc as plsc

N = 10000
E = 320000
D = 128

NC = 2
NS = 16
CH = 128
EPT = E // (NC * NS)
NCH = EPT // CH
TAIL = EPT - NCH * CH
NP = 10240
RPT = NP // NS
ZR = 128

BN = 1000


def _sc_degree(dst, zeros128, ones128):
    mesh = plsc.VectorSubcoreMesh(core_axis_name="c", subcore_axis_name="s")

    @functools.partial(
        pl.kernel,
        out_type=jax.ShapeDtypeStruct((NC, NP, D), jnp.float32),
        mesh=mesh,
        scratch_types=[
            pltpu.VMEM((CH,), jnp.int32),
            pltpu.VMEM((CH,), jnp.int32),
            pltpu.VMEM((TAIL,), jnp.int32),
            pltpu.VMEM((CH, D), jnp.float32),
            pltpu.VMEM_SHARED((NP, D), jnp.float32),
            pltpu.SemaphoreType.DMA,
            pltpu.SemaphoreType.DMA,
            pltpu.SemaphoreType.DMA,
            pltpu.SemaphoreType.DMA,
        ],
    )
    def deg_kernel(dst_hbm, z_hbm, ones_hbm, out_hbm,
                   dst_a, dst_b, dst_t, ones_v, acc,
                   zsem, isem, ssem_a, ssem_b):
        c = lax.axis_index("c")
        s = lax.axis_index("s")

        for r in range(0, RPT, ZR):
            pltpu.async_copy(z_hbm, acc.at[pl.ds(s * RPT + r, ZR)], zsem)
        pltpu.async_copy(ones_hbm, ones_v, isem)
        for r in range(0, RPT, ZR):
            pltpu.make_async_copy(z_hbm, acc.at[pl.ds(s * RPT + r, ZR)], zsem).wait()
        pltpu.make_async_copy(ones_hbm, ones_v, isem).wait()
        plsc.subcore_barrier()

        base = c * (E // NC) + s * EPT

        def chunk(e0, dst_v, ssem, first):
            if not first:
                pltpu.make_async_copy(ones_v, acc.at[dst_v], ssem).wait()
            pltpu.async_copy(dst_hbm.at[pl.ds(e0, CH)], dst_v, isem).wait()
            pltpu.async_copy(ones_v, acc.at[dst_v], ssem, add=True)

        chunk(base, dst_a, ssem_a, True)
        chunk(base + CH, dst_b, ssem_b, True)

        @pl.loop(2 * CH, NCH * CH, step=2 * CH)
        def _(e0):
            chunk(base + e0, dst_a, ssem_a, False)
            chunk(base + e0 + CH, dst_b, ssem_b, False)

        pltpu.async_copy(dst_hbm.at[pl.ds(base + NCH * CH, TAIL)], dst_t, isem).wait()
        pltpu.sync_copy(ones_v.at[pl.ds(0, TAIL)], acc.at[dst_t], add=True)

        pltpu.make_async_copy(ones_v, acc.at[dst_a], ssem_a).wait()
        pltpu.make_async_copy(ones_v, acc.at[dst_b], ssem_b).wait()
        plsc.subcore_barrier()

        for r in range(0, RPT, ZR):
            pltpu.async_copy(acc.at[pl.ds(s * RPT + r, ZR)],
                             out_hbm.at[c, pl.ds(s * RPT + r, ZR)], zsem)
        for r in range(0, RPT, ZR):
            pltpu.make_async_copy(acc.at[pl.ds(s * RPT + r, ZR)],
                                  out_hbm.at[c, pl.ds(s * RPT + r, ZR)], zsem).wait()

    return deg_kernel(dst, zeros128, ones128)


def _sc_hop(g, src, dst, zeros128):
    mesh = plsc.VectorSubcoreMesh(core_axis_name="c", subcore_axis_name="s")

    @functools.partial(
        pl.kernel,
        out_type=jax.ShapeDtypeStruct((NC, NP, D), jnp.float32),
        mesh=mesh,
        scratch_types=[
            pltpu.VMEM((CH,), jnp.int32),
            pltpu.VMEM((CH,), jnp.int32),
            pltpu.VMEM((CH, D), jnp.float32),
            pltpu.VMEM((CH,), jnp.int32),
            pltpu.VMEM((CH,), jnp.int32),
            pltpu.VMEM((CH, D), jnp.float32),
            pltpu.VMEM((TAIL,), jnp.int32),
            pltpu.VMEM((TAIL,), jnp.int32),
            pltpu.VMEM((TAIL, D), jnp.float32),
            pltpu.VMEM_SHARED((NP, D), jnp.float32),
            pltpu.SemaphoreType.DMA,
            pltpu.SemaphoreType.DMA,
            pltpu.SemaphoreType.DMA,
            pltpu.SemaphoreType.DMA,
            pltpu.SemaphoreType.DMA,
        ],
    )
    def hop_kernel(g_hbm, src_hbm, dst_hbm, z_hbm, out_hbm,
                   src_a, dst_a, rows_a, src_b, dst_b, rows_b,
                   src_t, dst_t, rows_t, acc,
                   zsem, isem, gsem, ssem_a, ssem_b):
        c = lax.axis_index("c")
        s = lax.axis_index("s")

        for r in range(0, RPT, ZR):
            pltpu.async_copy(z_hbm, acc.at[pl.ds(s * RPT + r, ZR)], zsem)
        for r in range(0, RPT, ZR):
            pltpu.make_async_copy(z_hbm, acc.at[pl.ds(s * RPT + r, ZR)], zsem).wait()
        plsc.subcore_barrier()

        base = c * (E // NC) + s * EPT

        def chunk(e0, src_v, dst_v, rows_v, ssem, first):
            if not first:
                pltpu.make_async_copy(rows_v, acc.at[dst_v], ssem).wait()
            pltpu.async_copy(src_hbm.at[pl.ds(e0, CH)], src_v, isem)
            pltpu.async_copy(dst_hbm.at[pl.ds(e0, CH)], dst_v, isem)
            pltpu.make_async_copy(src_hbm.at[pl.ds(e0, CH)], src_v, isem).wait()
            pltpu.make_async_copy(dst_hbm.at[pl.ds(e0, CH)], dst_v, isem).wait()
            pltpu.async_copy(g_hbm.at[src_v], rows_v, gsem).wait()
            pltpu.async_copy(rows_v, acc.at[dst_v], ssem, add=True)

        chunk(base, src_a, dst_a, rows_a, ssem_a, True)
        chunk(base + CH, src_b, dst_b, rows_b, ssem_b, True)

        @pl.loop(2 * CH, NCH * CH, step=2 * CH)
        def _(e0):
            chunk(base + e0, src_a, dst_a, rows_a, ssem_a, False)
            chunk(base + e0 + CH, src_b, dst_b, rows_b, ssem_b, False)

        pltpu.async_copy(src_hbm.at[pl.ds(base + NCH * CH, TAIL)], src_t, isem)
        pltpu.async_copy(dst_hbm.at[pl.ds(base + NCH * CH, TAIL)], dst_t, isem)
        pltpu.make_async_copy(src_hbm.at[pl.ds(base + NCH * CH, TAIL)], src_t, isem).wait()
        pltpu.make_async_copy(dst_hbm.at[pl.ds(base + NCH * CH, TAIL)], dst_t, isem).wait()
        pltpu.async_copy(g_hbm.at[src_t], rows_t, gsem).wait()
        pltpu.sync_copy(rows_t, acc.at[dst_t], add=True)

        pltpu.make_async_copy(rows_a, acc.at[dst_a], ssem_a).wait()
        pltpu.make_async_copy(rows_b, acc.at[dst_b], ssem_b).wait()
        plsc.subcore_barrier()

        for r in range(0, RPT, ZR):
            pltpu.async_copy(acc.at[pl.ds(s * RPT + r, ZR)],
                             out_hbm.at[c, pl.ds(s * RPT + r, ZR)], zsem)
        for r in range(0, RPT, ZR):
            pltpu.make_async_copy(acc.at[pl.ds(s * RPT + r, ZR)],
                                  out_hbm.at[c, pl.ds(s * RPT + r, ZR)], zsem).wait()

    return hop_kernel(g, src, dst, zeros128)


def _norm_from_deg(d_ref):
    deg = d_ref[0] + d_ref[1]
    return lax.rsqrt(jnp.maximum(deg, 1.0))[:, 0:1]


def _tc_scale(x, degp):
    def body(x_ref, d_ref, o_ref):
        o_ref[...] = x_ref[...] * _norm_from_deg(d_ref)

    return pl.pallas_call(
        body,
        grid=(N // BN,),
        in_specs=[
            pl.BlockSpec((BN, D), lambda i: (i, 0)),
            pl.BlockSpec((NC, BN, D), lambda i: (0, i, 0)),
        ],
        out_specs=pl.BlockSpec((BN, D), lambda i: (i, 0)),
        out_shape=jax.ShapeDtypeStruct((N, D), jnp.float32),
    )(x, degp)


def _tc_mid(aggp, degp):
    def body(a_ref, d_ref, h_ref, g_ref):
        nrm = _norm_from_deg(d_ref)
        h = (a_ref[0] + a_ref[1]) * nrm
        h_ref[...] = h
        g_ref[...] = h * nrm

    return pl.pallas_call(
        body,
        grid=(N // BN,),
        in_specs=[
            pl.BlockSpec((NC, BN, D), lambda i: (0, i, 0)),
            pl.BlockSpec((NC, BN, D), lambda i: (0, i, 0)),
        ],
        out_specs=[
            pl.BlockSpec((BN, D), lambda i: (i, 0)),
            pl.BlockSpec((BN, D), lambda i: (i, 0)),
        ],
        out_shape=[
            jax.ShapeDtypeStruct((N, D), jnp.float32),
            jax.ShapeDtypeStruct((N, D), jnp.float32),
        ],
    )(aggp, degp)


def _tc_layer1_end(f0, h1, aggp2, degp, Wt, b):
    def body(f0_ref, h1_ref, a_ref, d_ref, w_ref, b_ref, y_ref, g_ref):
        nrm = _norm_from_deg(d_ref)
        h2 = (a_ref[0] + a_ref[1]) * nrm
        cat = jnp.concatenate([f0_ref[...], h1_ref[...], h2], axis=1)
        y = jnp.dot(cat, w_ref[...], preferred_element_type=jnp.float32)
        y = jnp.maximum(y + b_ref[...], 0.0)
        y_ref[...] = y
        g_ref[...] = y * nrm

    return pl.pallas_call(
        body,
        grid=(N // BN,),
        in_specs=[
            pl.BlockSpec((BN, D), lambda i: (i, 0)),
            pl.BlockSpec((BN, D), lambda i: (i, 0)),
            pl.BlockSpec((NC, BN, D), lambda i: (0, i, 0)),
            pl.BlockSpec((NC, BN, D), lambda i: (0, i, 0)),
            pl.BlockSpec((3 * D, D), lambda i: (0, 0)),
            pl.BlockSpec((1, D), lambda i: (0, 0)),
        ],
        out_specs=[
            pl.BlockSpec((BN, D), lambda i: (i, 0)),
            pl.BlockSpec((BN, D), lambda i: (i, 0)),
        ],
        out_shape=[
            jax.ShapeDtypeStruct((N, D), jnp.float32),
            jax.ShapeDtypeStruct((N, D), jnp.float32),
        ],
    )(f0, h1, aggp2, degp, Wt, b)


def _tc_layer2_end(f0, h1, aggp2, degp, Wt, b):
    def body(f0_ref, h1_ref, a_ref, d_ref, w_ref, b_ref, o_ref):
        i = pl.program_id(0)

        @pl.when(i == 0)
        def _():
            o_ref[...] = jnp.zeros_like(o_ref)

        nrm = _norm_from_deg(d_ref)
        h2 = (a_ref[0] + a_ref[1]) * nrm
        cat = jnp.concatenate([f0_ref[...], h1_ref[...], h2], axis=1)
        y = jnp.dot(cat, w_ref[...], preferred_element_type=jnp.float32)
        y = jnp.maximum(y + b_ref[...], 0.0)
        o_ref[...] = jnp.maximum(o_ref[...], jnp.max(y, axis=0, keepdims=True))

    return pl.pallas_call(
        body,
        grid=(N // BN,),
        in_specs=[
            pl.BlockSpec((BN, D), lambda i: (i, 0)),
            pl.BlockSpec((BN, D), lambda i: (i, 0)),
            pl.BlockSpec((NC, BN, D), lambda i: (0, i, 0)),
            pl.BlockSpec((NC, BN, D), lambda i: (0, i, 0)),
            pl.BlockSpec((3 * D, D), lambda i: (0, 0)),
            pl.BlockSpec((1, D), lambda i: (0, 0)),
        ],
        out_specs=pl.BlockSpec((1, D), lambda i: (0, 0)),
        out_shape=jax.ShapeDtypeStruct((1, D), jnp.float32),
    )(f0, h1, aggp2, degp, Wt, b)


def kernel(x, edge_index, W1, b1, W2, b2):
    src = edge_index[0]
    dst = edge_index[1]
    zeros128 = jnp.zeros((ZR, D), jnp.float32)
    ones128 = jnp.ones((CH, D), jnp.float32)
    W1t = W1.T
    W2t = W2.T
    b1r = b1.reshape(1, D)
    b2r = b2.reshape(1, D)

    degp = _sc_degree(dst, zeros128, ones128)

    g0 = _tc_scale(x, degp)
    a1 = _sc_hop(g0, src, dst, zeros128)
    h1, g1 = _tc_mid(a1, degp)
    a2 = _sc_hop(g1, src, dst, zeros128)
    y1, gy1 = _tc_layer1_end(x, h1, a2, degp, W1t, b1r)

    a3 = _sc_hop(gy1, src, dst, zeros128)
    h1b, g1b = _tc_mid(a3, degp)
    a4 = _sc_hop(g1b, src, dst, zeros128)
    return _tc_layer2_end(y1, h1b, a4, degp, W2t, b2r)


# speedup vs baseline: 1.0016x; 1.0016x over previous
import functools

import jax
import jax.numpy as jnp
from jax import lax
from jax.experimental import pallas as pl
from jax.experimental.pallas import tpu as pltpu
from jax.experimental.pallas import tpu_sc as plsc

N = 10000
E = 320000
D = 128

NC = 2
NS = 16
CH = 128
EPT = E // (NC * NS)
NCH = EPT // CH
TAIL = EPT - NCH * CH
NP = 10240
RPT = NP // NS
ZR = 128

BN = 1000


def _sc_degree(dst, zeros128, ones128):
    mesh = plsc.VectorSubcoreMesh(core_axis_name="c", subcore_axis_name="s")

    @functools.partial(
        pl.kernel,
        out_type=jax.ShapeDtypeStruct((NC, NP, D), jnp.float32),
        mesh=mesh,
        scratch_types=[
            pltpu.VMEM((CH,), jnp.int32),
            pltpu.VMEM((CH,), jnp.int32),
            pltpu.VMEM((TAIL,), jnp.int32),
            pltpu.VMEM((CH, D), jnp.float32),
            pltpu.VMEM_SHARED((NP, D), jnp.float32),
            pltpu.SemaphoreType.DMA,
            pltpu.SemaphoreType.DMA,
            pltpu.SemaphoreType.DMA,
            pltpu.SemaphoreType.DMA,
        ],
    )
    def deg_kernel(dst_hbm, z_hbm, ones_hbm, out_hbm,
                   dst_a, dst_b, dst_t, ones_v, acc,
                   zsem, isem, ssem_a, ssem_b):
        c = lax.axis_index("c")
        s = lax.axis_index("s")

        for r in range(0, RPT, ZR):
            pltpu.async_copy(z_hbm, acc.at[pl.ds(s * RPT + r, ZR)], zsem)
        pltpu.async_copy(ones_hbm, ones_v, isem)
        for r in range(0, RPT, ZR):
            pltpu.make_async_copy(z_hbm, acc.at[pl.ds(s * RPT + r, ZR)], zsem).wait()
        pltpu.make_async_copy(ones_hbm, ones_v, isem).wait()
        plsc.subcore_barrier()

        base = c * (E // NC) + s * EPT

        def chunk(e0, dst_v, ssem, first):
            if not first:
                pltpu.make_async_copy(ones_v, acc.at[dst_v], ssem).wait()
            pltpu.async_copy(dst_hbm.at[pl.ds(e0, CH)], dst_v, isem).wait()
            pltpu.async_copy(ones_v, acc.at[dst_v], ssem, add=True)

        chunk(base, dst_a, ssem_a, True)
        chunk(base + CH, dst_b, ssem_b, True)

        @pl.loop(2 * CH, NCH * CH, step=2 * CH)
        def _(e0):
            chunk(base + e0, dst_a, ssem_a, False)
            chunk(base + e0 + CH, dst_b, ssem_b, False)

        pltpu.async_copy(dst_hbm.at[pl.ds(base + NCH * CH, TAIL)], dst_t, isem).wait()
        pltpu.sync_copy(ones_v.at[pl.ds(0, TAIL)], acc.at[dst_t], add=True)

        pltpu.make_async_copy(ones_v, acc.at[dst_a], ssem_a).wait()
        pltpu.make_async_copy(ones_v, acc.at[dst_b], ssem_b).wait()
        plsc.subcore_barrier()

        for r in range(0, RPT, ZR):
            pltpu.async_copy(acc.at[pl.ds(s * RPT + r, ZR)],
                             out_hbm.at[c, pl.ds(s * RPT + r, ZR)], zsem)
        for r in range(0, RPT, ZR):
            pltpu.make_async_copy(acc.at[pl.ds(s * RPT + r, ZR)],
                                  out_hbm.at[c, pl.ds(s * RPT + r, ZR)], zsem).wait()

    return deg_kernel(dst, zeros128, ones128)


def _sc_hop(g, src, dst, zeros128):
    mesh = plsc.VectorSubcoreMesh(core_axis_name="c", subcore_axis_name="s")

    @functools.partial(
        pl.kernel,
        out_type=jax.ShapeDtypeStruct((NC, NP, D), jnp.float32),
        mesh=mesh,
        scratch_types=[
            pltpu.VMEM((CH,), jnp.int32),
            pltpu.VMEM((CH,), jnp.int32),
            pltpu.VMEM((CH, D), jnp.float32),
            pltpu.VMEM((CH,), jnp.int32),
            pltpu.VMEM((CH,), jnp.int32),
            pltpu.VMEM((CH, D), jnp.float32),
            pltpu.VMEM((TAIL,), jnp.int32),
            pltpu.VMEM((TAIL,), jnp.int32),
            pltpu.VMEM((TAIL, D), jnp.float32),
            pltpu.VMEM_SHARED((NP, D), jnp.float32),
            pltpu.SemaphoreType.DMA,
            pltpu.SemaphoreType.DMA,
            pltpu.SemaphoreType.DMA,
            pltpu.SemaphoreType.DMA,
            pltpu.SemaphoreType.DMA,
        ],
    )
    def hop_kernel(g_hbm, src_hbm, dst_hbm, z_hbm, out_hbm,
                   src_a, dst_a, rows_a, src_b, dst_b, rows_b,
                   src_t, dst_t, rows_t, acc,
                   zsem, isem, gsem, ssem_a, ssem_b):
        c = lax.axis_index("c")
        s = lax.axis_index("s")

        for r in range(0, RPT, ZR):
            pltpu.async_copy(z_hbm, acc.at[pl.ds(s * RPT + r, ZR)], zsem)
        for r in range(0, RPT, ZR):
            pltpu.make_async_copy(z_hbm, acc.at[pl.ds(s * RPT + r, ZR)], zsem).wait()
        plsc.subcore_barrier()

        base = c * (E // NC) + s * EPT

        def idx_issue(e0, src_v, dst_v):
            pltpu.async_copy(src_hbm.at[pl.ds(e0, CH)], src_v, isem)
            pltpu.async_copy(dst_hbm.at[pl.ds(e0, CH)], dst_v, isem)

        def idx_wait(e0, src_v, dst_v):
            pltpu.make_async_copy(src_hbm.at[pl.ds(e0, CH)], src_v, isem).wait()
            pltpu.make_async_copy(dst_hbm.at[pl.ds(e0, CH)], dst_v, isem).wait()

        def chunk(e0, src_v, dst_v, rows_v, ssem,
                  o_src, o_dst, o_rows, o_ssem, drain, prefetch):
            idx_wait(e0, src_v, dst_v)
            pltpu.async_copy(g_hbm.at[src_v], rows_v, gsem).wait()
            if drain:
                pltpu.make_async_copy(o_rows, acc.at[o_dst], o_ssem).wait()
            if prefetch:
                idx_issue(e0 + CH, o_src, o_dst)
            pltpu.async_copy(rows_v, acc.at[dst_v], ssem, add=True)

        A = (src_a, dst_a, rows_a, ssem_a)
        B = (src_b, dst_b, rows_b, ssem_b)

        idx_issue(base, src_a, dst_a)
        chunk(base, *A, *B, False, True)
        chunk(base + CH, *B, *A, True, True)

        @pl.loop(2 * CH, (NCH - 2) * CH, step=2 * CH)
        def _(e0):
            chunk(base + e0, *A, *B, True, True)
            chunk(base + e0 + CH, *B, *A, True, True)

        chunk(base + (NCH - 2) * CH, *A, *B, True, True)
        chunk(base + (NCH - 1) * CH, *B, *A, True, False)

        pltpu.async_copy(src_hbm.at[pl.ds(base + NCH * CH, TAIL)], src_t, isem)
        pltpu.async_copy(dst_hbm.at[pl.ds(base + NCH * CH, TAIL)], dst_t, isem)
        pltpu.make_async_copy(src_hbm.at[pl.ds(base + NCH * CH, TAIL)], src_t, isem).wait()
        pltpu.make_async_copy(dst_hbm.at[pl.ds(base + NCH * CH, TAIL)], dst_t, isem).wait()
        pltpu.async_copy(g_hbm.at[src_t], rows_t, gsem).wait()
        pltpu.sync_copy(rows_t, acc.at[dst_t], add=True)

        pltpu.make_async_copy(rows_b, acc.at[dst_b], ssem_b).wait()
        plsc.subcore_barrier()

        for r in range(0, RPT, ZR):
            pltpu.async_copy(acc.at[pl.ds(s * RPT + r, ZR)],
                             out_hbm.at[c, pl.ds(s * RPT + r, ZR)], zsem)
        for r in range(0, RPT, ZR):
            pltpu.make_async_copy(acc.at[pl.ds(s * RPT + r, ZR)],
                                  out_hbm.at[c, pl.ds(s * RPT + r, ZR)], zsem).wait()

    return hop_kernel(g, src, dst, zeros128)


def _norm_from_deg(d_ref):
    deg = d_ref[0] + d_ref[1]
    return lax.rsqrt(jnp.maximum(deg, 1.0))[:, 0:1]


def _tc_scale(x, degp):
    def body(x_ref, d_ref, o_ref):
        o_ref[...] = x_ref[...] * _norm_from_deg(d_ref)

    return pl.pallas_call(
        body,
        grid=(N // BN,),
        in_specs=[
            pl.BlockSpec((BN, D), lambda i: (i, 0)),
            pl.BlockSpec((NC, BN, D), lambda i: (0, i, 0)),
        ],
        out_specs=pl.BlockSpec((BN, D), lambda i: (i, 0)),
        out_shape=jax.ShapeDtypeStruct((N, D), jnp.float32),
    )(x, degp)


def _tc_mid(aggp, degp):
    def body(a_ref, d_ref, h_ref, g_ref):
        nrm = _norm_from_deg(d_ref)
        h = (a_ref[0] + a_ref[1]) * nrm
        h_ref[...] = h
        g_ref[...] = h * nrm

    return pl.pallas_call(
        body,
        grid=(N // BN,),
        in_specs=[
            pl.BlockSpec((NC, BN, D), lambda i: (0, i, 0)),
            pl.BlockSpec((NC, BN, D), lambda i: (0, i, 0)),
        ],
        out_specs=[
            pl.BlockSpec((BN, D), lambda i: (i, 0)),
            pl.BlockSpec((BN, D), lambda i: (i, 0)),
        ],
        out_shape=[
            jax.ShapeDtypeStruct((N, D), jnp.float32),
            jax.ShapeDtypeStruct((N, D), jnp.float32),
        ],
    )(aggp, degp)


def _tc_layer1_end(f0, h1, aggp2, degp, Wt, b):
    def body(f0_ref, h1_ref, a_ref, d_ref, w_ref, b_ref, y_ref, g_ref):
        nrm = _norm_from_deg(d_ref)
        h2 = (a_ref[0] + a_ref[1]) * nrm
        cat = jnp.concatenate([f0_ref[...], h1_ref[...], h2], axis=1)
        y = jnp.dot(cat, w_ref[...], preferred_element_type=jnp.float32)
        y = jnp.maximum(y + b_ref[...], 0.0)
        y_ref[...] = y
        g_ref[...] = y * nrm

    return pl.pallas_call(
        body,
        grid=(N // BN,),
        in_specs=[
            pl.BlockSpec((BN, D), lambda i: (i, 0)),
            pl.BlockSpec((BN, D), lambda i: (i, 0)),
            pl.BlockSpec((NC, BN, D), lambda i: (0, i, 0)),
            pl.BlockSpec((NC, BN, D), lambda i: (0, i, 0)),
            pl.BlockSpec((3 * D, D), lambda i: (0, 0)),
            pl.BlockSpec((1, D), lambda i: (0, 0)),
        ],
        out_specs=[
            pl.BlockSpec((BN, D), lambda i: (i, 0)),
            pl.BlockSpec((BN, D), lambda i: (i, 0)),
        ],
        out_shape=[
            jax.ShapeDtypeStruct((N, D), jnp.float32),
            jax.ShapeDtypeStruct((N, D), jnp.float32),
        ],
    )(f0, h1, aggp2, degp, Wt, b)


def _tc_layer2_end(f0, h1, aggp2, degp, Wt, b):
    def body(f0_ref, h1_ref, a_ref, d_ref, w_ref, b_ref, o_ref):
        i = pl.program_id(0)

        @pl.when(i == 0)
        def _():
            o_ref[...] = jnp.zeros_like(o_ref)

        nrm = _norm_from_deg(d_ref)
        h2 = (a_ref[0] + a_ref[1]) * nrm
        cat = jnp.concatenate([f0_ref[...], h1_ref[...], h2], axis=1)
        y = jnp.dot(cat, w_ref[...], preferred_element_type=jnp.float32)
        y = jnp.maximum(y + b_ref[...], 0.0)
        o_ref[...] = jnp.maximum(o_ref[...], jnp.max(y, axis=0, keepdims=True))

    return pl.pallas_call(
        body,
        grid=(N // BN,),
        in_specs=[
            pl.BlockSpec((BN, D), lambda i: (i, 0)),
            pl.BlockSpec((BN, D), lambda i: (i, 0)),
            pl.BlockSpec((NC, BN, D), lambda i: (0, i, 0)),
            pl.BlockSpec((NC, BN, D), lambda i: (0, i, 0)),
            pl.BlockSpec((3 * D, D), lambda i: (0, 0)),
            pl.BlockSpec((1, D), lambda i: (0, 0)),
        ],
        out_specs=pl.BlockSpec((1, D), lambda i: (0, 0)),
        out_shape=jax.ShapeDtypeStruct((1, D), jnp.float32),
    )(f0, h1, aggp2, degp, Wt, b)


def kernel(x, edge_index, W1, b1, W2, b2):
    src = edge_index[0]
    dst = edge_index[1]
    zeros128 = jnp.zeros((ZR, D), jnp.float32)
    ones128 = jnp.ones((CH, D), jnp.float32)
    W1t = W1.T
    W2t = W2.T
    b1r = b1.reshape(1, D)
    b2r = b2.reshape(1, D)

    degp = _sc_degree(dst, zeros128, ones128)

    g0 = _tc_scale(x, degp)
    a1 = _sc_hop(g0, src, dst, zeros128)
    h1, g1 = _tc_mid(a1, degp)
    a2 = _sc_hop(g1, src, dst, zeros128)
    y1, gy1 = _tc_layer1_end(x, h1, a2, degp, W1t, b1r)

    a3 = _sc_hop(gy1, src, dst, zeros128)
    h1b, g1b = _tc_mid(a3, degp)
    a4 = _sc_hop(g1b, src, dst, zeros128)
    return _tc_layer2_end(y1, h1b, a4, degp, W2t, b2r)


# speedup vs baseline: 1.0198x; 1.0182x over previous
import functools

import jax
import jax.numpy as jnp
from jax import lax
from jax.experimental import pallas as pl
from jax.experimental.pallas import tpu as pltpu
from jax.experimental.pallas import tpu_sc as plsc

N = 10000
E = 320000
D = 128

NC = 2
NS = 16
CH = 128
EPT = E // (NC * NS)
NCH = EPT // CH
TAIL = EPT - NCH * CH
NP = 10240
RPT = NP // NS
ZR = 128

BN = 1000


def _sc_degree(dst, zeros128, ones128):
    mesh = plsc.VectorSubcoreMesh(core_axis_name="c", subcore_axis_name="s")

    @functools.partial(
        pl.kernel,
        out_type=jax.ShapeDtypeStruct((NC, NP, D), jnp.float32),
        mesh=mesh,
        scratch_types=[
            pltpu.VMEM((CH,), jnp.int32),
            pltpu.VMEM((CH,), jnp.int32),
            pltpu.VMEM((TAIL,), jnp.int32),
            pltpu.VMEM((CH, D), jnp.float32),
            pltpu.VMEM_SHARED((NP, D), jnp.float32),
            pltpu.SemaphoreType.DMA,
            pltpu.SemaphoreType.DMA,
            pltpu.SemaphoreType.DMA,
            pltpu.SemaphoreType.DMA,
        ],
    )
    def deg_kernel(dst_hbm, z_hbm, ones_hbm, out_hbm,
                   dst_a, dst_b, dst_t, ones_v, acc,
                   zsem, isem, ssem_a, ssem_b):
        c = lax.axis_index("c")
        s = lax.axis_index("s")

        for r in range(0, RPT, ZR):
            pltpu.async_copy(z_hbm, acc.at[pl.ds(s * RPT + r, ZR)], zsem)
        pltpu.async_copy(ones_hbm, ones_v, isem)
        for r in range(0, RPT, ZR):
            pltpu.make_async_copy(z_hbm, acc.at[pl.ds(s * RPT + r, ZR)], zsem).wait()
        pltpu.make_async_copy(ones_hbm, ones_v, isem).wait()
        plsc.subcore_barrier()

        base = c * (E // NC) + s * EPT

        def chunk(e0, dst_v, ssem, first):
            if not first:
                pltpu.make_async_copy(ones_v, acc.at[dst_v], ssem).wait()
            pltpu.async_copy(dst_hbm.at[pl.ds(e0, CH)], dst_v, isem).wait()
            pltpu.async_copy(ones_v, acc.at[dst_v], ssem, add=True)

        chunk(base, dst_a, ssem_a, True)
        chunk(base + CH, dst_b, ssem_b, True)

        @pl.loop(2 * CH, NCH * CH, step=2 * CH)
        def _(e0):
            chunk(base + e0, dst_a, ssem_a, False)
            chunk(base + e0 + CH, dst_b, ssem_b, False)

        pltpu.async_copy(dst_hbm.at[pl.ds(base + NCH * CH, TAIL)], dst_t, isem).wait()
        pltpu.sync_copy(ones_v.at[pl.ds(0, TAIL)], acc.at[dst_t], add=True)

        pltpu.make_async_copy(ones_v, acc.at[dst_a], ssem_a).wait()
        pltpu.make_async_copy(ones_v, acc.at[dst_b], ssem_b).wait()
        plsc.subcore_barrier()

        for r in range(0, RPT, ZR):
            pltpu.async_copy(acc.at[pl.ds(s * RPT + r, ZR)],
                             out_hbm.at[c, pl.ds(s * RPT + r, ZR)], zsem)
        for r in range(0, RPT, ZR):
            pltpu.make_async_copy(acc.at[pl.ds(s * RPT + r, ZR)],
                                  out_hbm.at[c, pl.ds(s * RPT + r, ZR)], zsem).wait()

    return deg_kernel(dst, zeros128, ones128)


def _sc_hop(g, src, dst, zeros128):
    mesh = plsc.VectorSubcoreMesh(core_axis_name="c", subcore_axis_name="s")

    @functools.partial(
        pl.kernel,
        out_type=jax.ShapeDtypeStruct((NC, NP, D), jnp.float32),
        mesh=mesh,
        scratch_types=[
            pltpu.VMEM((CH // 2,), jnp.int32),
            pltpu.VMEM((CH // 2,), jnp.int32),
            pltpu.VMEM((CH,), jnp.int32),
            pltpu.VMEM((CH, D), jnp.float32),
            pltpu.VMEM((CH // 2,), jnp.int32),
            pltpu.VMEM((CH // 2,), jnp.int32),
            pltpu.VMEM((CH,), jnp.int32),
            pltpu.VMEM((CH, D), jnp.float32),
            pltpu.VMEM((TAIL,), jnp.int32),
            pltpu.VMEM((TAIL,), jnp.int32),
            pltpu.VMEM((TAIL, D), jnp.float32),
            pltpu.VMEM_SHARED((NP, D), jnp.float32),
            pltpu.SemaphoreType.DMA,
            pltpu.SemaphoreType.DMA,
            pltpu.SemaphoreType.DMA,
            pltpu.SemaphoreType.DMA,
            pltpu.SemaphoreType.DMA,
            pltpu.SemaphoreType.DMA,
        ],
    )
    def hop_kernel(g_hbm, src_hbm, dst_hbm, z_hbm, out_hbm,
                   src_a1, src_a2, dst_a, rows_a, src_b1, src_b2, dst_b, rows_b,
                   src_t, dst_t, rows_t, acc,
                   zsem, isem, gsem1, gsem2, ssem_a, ssem_b):
        c = lax.axis_index("c")
        s = lax.axis_index("s")

        for r in range(0, RPT, ZR):
            pltpu.async_copy(z_hbm, acc.at[pl.ds(s * RPT + r, ZR)], zsem)
        for r in range(0, RPT, ZR):
            pltpu.make_async_copy(z_hbm, acc.at[pl.ds(s * RPT + r, ZR)], zsem).wait()
        plsc.subcore_barrier()

        base = c * (E // NC) + s * EPT

        H = CH // 2

        def idx_issue(e0, src_v1, src_v2, dst_v):
            pltpu.async_copy(src_hbm.at[pl.ds(e0, H)], src_v1, isem)
            pltpu.async_copy(src_hbm.at[pl.ds(e0 + H, H)], src_v2, isem)
            pltpu.async_copy(dst_hbm.at[pl.ds(e0, CH)], dst_v, isem)

        def idx_wait(e0, src_v1, src_v2, dst_v):
            pltpu.make_async_copy(src_hbm.at[pl.ds(e0, H)], src_v1, isem).wait()
            pltpu.make_async_copy(src_hbm.at[pl.ds(e0 + H, H)], src_v2, isem).wait()
            pltpu.make_async_copy(dst_hbm.at[pl.ds(e0, CH)], dst_v, isem).wait()

        def chunk(e0, src_v1, src_v2, dst_v, rows_v, ssem,
                  o_src1, o_src2, o_dst, o_rows, o_ssem, drain, prefetch):
            idx_wait(e0, src_v1, src_v2, dst_v)
            pltpu.async_copy(g_hbm.at[src_v1], rows_v.at[pl.ds(0, H)], gsem1)
            pltpu.async_copy(g_hbm.at[src_v2], rows_v.at[pl.ds(H, H)], gsem2)
            pltpu.make_async_copy(g_hbm.at[src_v1], rows_v.at[pl.ds(0, H)], gsem1).wait()
            pltpu.make_async_copy(g_hbm.at[src_v2], rows_v.at[pl.ds(H, H)], gsem2).wait()
            if drain:
                pltpu.make_async_copy(o_rows, acc.at[o_dst], o_ssem).wait()
            if prefetch:
                idx_issue(e0 + CH, o_src1, o_src2, o_dst)
            pltpu.async_copy(rows_v, acc.at[dst_v], ssem, add=True)

        A = (src_a1, src_a2, dst_a, rows_a, ssem_a)
        B = (src_b1, src_b2, dst_b, rows_b, ssem_b)

        idx_issue(base, src_a1, src_a2, dst_a)
        chunk(base, *A, *B, False, True)
        chunk(base + CH, *B, *A, True, True)

        @pl.loop(2 * CH, (NCH - 2) * CH, step=2 * CH)
        def _(e0):
            chunk(base + e0, *A, *B, True, True)
            chunk(base + e0 + CH, *B, *A, True, True)

        chunk(base + (NCH - 2) * CH, *A, *B, True, True)
        chunk(base + (NCH - 1) * CH, *B, *A, True, False)

        pltpu.async_copy(src_hbm.at[pl.ds(base + NCH * CH, TAIL)], src_t, isem)
        pltpu.async_copy(dst_hbm.at[pl.ds(base + NCH * CH, TAIL)], dst_t, isem)
        pltpu.make_async_copy(src_hbm.at[pl.ds(base + NCH * CH, TAIL)], src_t, isem).wait()
        pltpu.make_async_copy(dst_hbm.at[pl.ds(base + NCH * CH, TAIL)], dst_t, isem).wait()
        pltpu.async_copy(g_hbm.at[src_t], rows_t, gsem1).wait()
        pltpu.sync_copy(rows_t, acc.at[dst_t], add=True)

        pltpu.make_async_copy(rows_b, acc.at[dst_b], ssem_b).wait()
        plsc.subcore_barrier()

        for r in range(0, RPT, ZR):
            pltpu.async_copy(acc.at[pl.ds(s * RPT + r, ZR)],
                             out_hbm.at[c, pl.ds(s * RPT + r, ZR)], zsem)
        for r in range(0, RPT, ZR):
            pltpu.make_async_copy(acc.at[pl.ds(s * RPT + r, ZR)],
                                  out_hbm.at[c, pl.ds(s * RPT + r, ZR)], zsem).wait()

    return hop_kernel(g, src, dst, zeros128)


def _norm_from_deg(d_ref):
    deg = d_ref[0] + d_ref[1]
    return lax.rsqrt(jnp.maximum(deg, 1.0))[:, 0:1]


def _tc_scale(x, degp):
    def body(x_ref, d_ref, o_ref):
        o_ref[...] = x_ref[...] * _norm_from_deg(d_ref)

    return pl.pallas_call(
        body,
        grid=(N // BN,),
        in_specs=[
            pl.BlockSpec((BN, D), lambda i: (i, 0)),
            pl.BlockSpec((NC, BN, D), lambda i: (0, i, 0)),
        ],
        out_specs=pl.BlockSpec((BN, D), lambda i: (i, 0)),
        out_shape=jax.ShapeDtypeStruct((N, D), jnp.float32),
    )(x, degp)


def _tc_mid(aggp, degp):
    def body(a_ref, d_ref, h_ref, g_ref):
        nrm = _norm_from_deg(d_ref)
        h = (a_ref[0] + a_ref[1]) * nrm
        h_ref[...] = h
        g_ref[...] = h * nrm

    return pl.pallas_call(
        body,
        grid=(N // BN,),
        in_specs=[
            pl.BlockSpec((NC, BN, D), lambda i: (0, i, 0)),
            pl.BlockSpec((NC, BN, D), lambda i: (0, i, 0)),
        ],
        out_specs=[
            pl.BlockSpec((BN, D), lambda i: (i, 0)),
            pl.BlockSpec((BN, D), lambda i: (i, 0)),
        ],
        out_shape=[
            jax.ShapeDtypeStruct((N, D), jnp.float32),
            jax.ShapeDtypeStruct((N, D), jnp.float32),
        ],
    )(aggp, degp)


def _tc_layer1_end(f0, h1, aggp2, degp, Wt, b):
    def body(f0_ref, h1_ref, a_ref, d_ref, w_ref, b_ref, y_ref, g_ref):
        nrm = _norm_from_deg(d_ref)
        h2 = (a_ref[0] + a_ref[1]) * nrm
        cat = jnp.concatenate([f0_ref[...], h1_ref[...], h2], axis=1)
        y = jnp.dot(cat, w_ref[...], preferred_element_type=jnp.float32)
        y = jnp.maximum(y + b_ref[...], 0.0)
        y_ref[...] = y
        g_ref[...] = y * nrm

    return pl.pallas_call(
        body,
        grid=(N // BN,),
        in_specs=[
            pl.BlockSpec((BN, D), lambda i: (i, 0)),
            pl.BlockSpec((BN, D), lambda i: (i, 0)),
            pl.BlockSpec((NC, BN, D), lambda i: (0, i, 0)),
            pl.BlockSpec((NC, BN, D), lambda i: (0, i, 0)),
            pl.BlockSpec((3 * D, D), lambda i: (0, 0)),
            pl.BlockSpec((1, D), lambda i: (0, 0)),
        ],
        out_specs=[
            pl.BlockSpec((BN, D), lambda i: (i, 0)),
            pl.BlockSpec((BN, D), lambda i: (i, 0)),
        ],
        out_shape=[
            jax.ShapeDtypeStruct((N, D), jnp.float32),
            jax.ShapeDtypeStruct((N, D), jnp.float32),
        ],
    )(f0, h1, aggp2, degp, Wt, b)


def _tc_layer2_end(f0, h1, aggp2, degp, Wt, b):
    def body(f0_ref, h1_ref, a_ref, d_ref, w_ref, b_ref, o_ref):
        i = pl.program_id(0)

        @pl.when(i == 0)
        def _():
            o_ref[...] = jnp.zeros_like(o_ref)

        nrm = _norm_from_deg(d_ref)
        h2 = (a_ref[0] + a_ref[1]) * nrm
        cat = jnp.concatenate([f0_ref[...], h1_ref[...], h2], axis=1)
        y = jnp.dot(cat, w_ref[...], preferred_element_type=jnp.float32)
        y = jnp.maximum(y + b_ref[...], 0.0)
        o_ref[...] = jnp.maximum(o_ref[...], jnp.max(y, axis=0, keepdims=True))

    return pl.pallas_call(
        body,
        grid=(N // BN,),
        in_specs=[
            pl.BlockSpec((BN, D), lambda i: (i, 0)),
            pl.BlockSpec((BN, D), lambda i: (i, 0)),
            pl.BlockSpec((NC, BN, D), lambda i: (0, i, 0)),
            pl.BlockSpec((NC, BN, D), lambda i: (0, i, 0)),
            pl.BlockSpec((3 * D, D), lambda i: (0, 0)),
            pl.BlockSpec((1, D), lambda i: (0, 0)),
        ],
        out_specs=pl.BlockSpec((1, D), lambda i: (0, 0)),
        out_shape=jax.ShapeDtypeStruct((1, D), jnp.float32),
    )(f0, h1, aggp2, degp, Wt, b)


def kernel(x, edge_index, W1, b1, W2, b2):
    src = edge_index[0]
    dst = edge_index[1]
    zeros128 = jnp.zeros((ZR, D), jnp.float32)
    ones128 = jnp.ones((CH, D), jnp.float32)
    W1t = W1.T
    W2t = W2.T
    b1r = b1.reshape(1, D)
    b2r = b2.reshape(1, D)

    degp = _sc_degree(dst, zeros128, ones128)

    g0 = _tc_scale(x, degp)
    a1 = _sc_hop(g0, src, dst, zeros128)
    h1, g1 = _tc_mid(a1, degp)
    a2 = _sc_hop(g1, src, dst, zeros128)
    y1, gy1 = _tc_layer1_end(x, h1, a2, degp, W1t, b1r)

    a3 = _sc_hop(gy1, src, dst, zeros128)
    h1b, g1b = _tc_mid(a3, degp)
    a4 = _sc_hop(g1b, src, dst, zeros128)
    return _tc_layer2_end(y1, h1b, a4, degp, W2t, b2r)


# speedup vs baseline: 1.0220x; 1.0022x over previous
import functools

import jax
import jax.numpy as jnp
from jax import lax
from jax.experimental import pallas as pl
from jax.experimental.pallas import tpu as pltpu
from jax.experimental.pallas import tpu_sc as plsc

N = 10000
E = 320000
D = 128

NC = 2
NS = 16
CH = 128
EPT = E // (NC * NS)
NCH = EPT // CH
TAIL = EPT - NCH * CH
NP = 10240
RPT = NP // NS
ZR = 128

BN = 1000


def _sc_degree(dst, zeros128, ones128):
    mesh = plsc.VectorSubcoreMesh(core_axis_name="c", subcore_axis_name="s")

    @functools.partial(
        pl.kernel,
        out_type=jax.ShapeDtypeStruct((NC, NP, D), jnp.float32),
        mesh=mesh,
        scratch_types=[
            pltpu.VMEM((CH,), jnp.int32),
            pltpu.VMEM((CH,), jnp.int32),
            pltpu.VMEM((TAIL,), jnp.int32),
            pltpu.VMEM((CH, D), jnp.float32),
            pltpu.VMEM_SHARED((NP, D), jnp.float32),
            pltpu.SemaphoreType.DMA,
            pltpu.SemaphoreType.DMA,
            pltpu.SemaphoreType.DMA,
            pltpu.SemaphoreType.DMA,
        ],
    )
    def deg_kernel(dst_hbm, z_hbm, ones_hbm, out_hbm,
                   dst_a, dst_b, dst_t, ones_v, acc,
                   zsem, isem, ssem_a, ssem_b):
        c = lax.axis_index("c")
        s = lax.axis_index("s")

        for r in range(0, RPT, ZR):
            pltpu.async_copy(z_hbm, acc.at[pl.ds(s * RPT + r, ZR)], zsem)
        pltpu.async_copy(ones_hbm, ones_v, isem)
        for r in range(0, RPT, ZR):
            pltpu.make_async_copy(z_hbm, acc.at[pl.ds(s * RPT + r, ZR)], zsem).wait()
        pltpu.make_async_copy(ones_hbm, ones_v, isem).wait()
        plsc.subcore_barrier()

        base = c * (E // NC) + s * EPT

        def chunk(e0, dst_v, ssem, first):
            if not first:
                pltpu.make_async_copy(ones_v, acc.at[dst_v], ssem).wait()
            pltpu.async_copy(dst_hbm.at[pl.ds(e0, CH)], dst_v, isem).wait()
            pltpu.async_copy(ones_v, acc.at[dst_v], ssem, add=True)

        chunk(base, dst_a, ssem_a, True)
        chunk(base + CH, dst_b, ssem_b, True)

        @pl.loop(2 * CH, NCH * CH, step=2 * CH)
        def _(e0):
            chunk(base + e0, dst_a, ssem_a, False)
            chunk(base + e0 + CH, dst_b, ssem_b, False)

        pltpu.async_copy(dst_hbm.at[pl.ds(base + NCH * CH, TAIL)], dst_t, isem).wait()
        pltpu.sync_copy(ones_v.at[pl.ds(0, TAIL)], acc.at[dst_t], add=True)

        pltpu.make_async_copy(ones_v, acc.at[dst_a], ssem_a).wait()
        pltpu.make_async_copy(ones_v, acc.at[dst_b], ssem_b).wait()
        plsc.subcore_barrier()

        for r in range(0, RPT, ZR):
            pltpu.async_copy(acc.at[pl.ds(s * RPT + r, ZR)],
                             out_hbm.at[c, pl.ds(s * RPT + r, ZR)], zsem)
        for r in range(0, RPT, ZR):
            pltpu.make_async_copy(acc.at[pl.ds(s * RPT + r, ZR)],
                                  out_hbm.at[c, pl.ds(s * RPT + r, ZR)], zsem).wait()

    return deg_kernel(dst, zeros128, ones128)


def _sc_hop(g, src, dst, zeros128):
    mesh = plsc.VectorSubcoreMesh(core_axis_name="c", subcore_axis_name="s")

    @functools.partial(
        pl.kernel,
        out_type=jax.ShapeDtypeStruct((NC, NP, D), jnp.float32),
        mesh=mesh,
        scratch_types=[
            pltpu.VMEM((CH // 2,), jnp.int32),
            pltpu.VMEM((CH // 2,), jnp.int32),
            pltpu.VMEM((CH,), jnp.int32),
            pltpu.VMEM((CH, D), jnp.float32),
            pltpu.VMEM((CH // 2,), jnp.int32),
            pltpu.VMEM((CH // 2,), jnp.int32),
            pltpu.VMEM((CH,), jnp.int32),
            pltpu.VMEM((CH, D), jnp.float32),
            pltpu.VMEM((TAIL,), jnp.int32),
            pltpu.VMEM((TAIL,), jnp.int32),
            pltpu.VMEM((TAIL, D), jnp.float32),
            pltpu.VMEM_SHARED((NP, D), jnp.float32),
            pltpu.SemaphoreType.DMA,
            pltpu.SemaphoreType.DMA,
            pltpu.SemaphoreType.DMA,
            pltpu.SemaphoreType.DMA,
            pltpu.SemaphoreType.DMA,
            pltpu.SemaphoreType.DMA,
        ],
    )
    def hop_kernel(g_hbm, src_hbm, dst_hbm, z_hbm, out_hbm,
                   src_a1, src_a2, dst_a, rows_a, src_b1, src_b2, dst_b, rows_b,
                   src_t, dst_t, rows_t, acc,
                   zsem, isem, gsem1, gsem2, ssem_a, ssem_b):
        c = lax.axis_index("c")
        s = lax.axis_index("s")

        for r in range(0, RPT, ZR):
            pltpu.async_copy(z_hbm, acc.at[pl.ds(s * RPT + r, ZR)], zsem)
        for r in range(0, RPT, ZR):
            pltpu.make_async_copy(z_hbm, acc.at[pl.ds(s * RPT + r, ZR)], zsem).wait()
        plsc.subcore_barrier()

        base = c * (E // NC) + s * EPT

        H = CH // 2

        def idx_issue(e0, src_v1, src_v2, dst_v):
            pltpu.async_copy(src_hbm.at[pl.ds(e0, H)], src_v1, isem)
            pltpu.async_copy(src_hbm.at[pl.ds(e0 + H, H)], src_v2, isem)
            pltpu.async_copy(dst_hbm.at[pl.ds(e0, CH)], dst_v, isem)

        def idx_wait(e0, src_v1, src_v2, dst_v):
            pltpu.make_async_copy(src_hbm.at[pl.ds(e0, H)], src_v1, isem).wait()
            pltpu.make_async_copy(src_hbm.at[pl.ds(e0 + H, H)], src_v2, isem).wait()
            pltpu.make_async_copy(dst_hbm.at[pl.ds(e0, CH)], dst_v, isem).wait()

        def chunk(e0, src_v1, src_v2, dst_v, rows_v, ssem,
                  o_src1, o_src2, o_dst, o_rows, o_ssem, drain, prefetch):
            idx_wait(e0, src_v1, src_v2, dst_v)
            pltpu.async_copy(g_hbm.at[src_v1], rows_v.at[pl.ds(0, H)], gsem1)
            pltpu.async_copy(g_hbm.at[src_v2], rows_v.at[pl.ds(H, H)], gsem2)
            pltpu.make_async_copy(g_hbm.at[src_v1], rows_v.at[pl.ds(0, H)], gsem1).wait()
            pltpu.make_async_copy(g_hbm.at[src_v2], rows_v.at[pl.ds(H, H)], gsem2).wait()
            pltpu.async_copy(rows_v, acc.at[dst_v], ssem, add=True)
            if drain:
                pltpu.make_async_copy(o_rows, acc.at[o_dst], o_ssem).wait()
            if prefetch:
                idx_issue(e0 + CH, o_src1, o_src2, o_dst)

        A = (src_a1, src_a2, dst_a, rows_a, ssem_a)
        B = (src_b1, src_b2, dst_b, rows_b, ssem_b)

        idx_issue(base, src_a1, src_a2, dst_a)
        chunk(base, *A, *B, False, True)
        chunk(base + CH, *B, *A, True, True)

        @pl.loop(2 * CH, (NCH - 2) * CH, step=2 * CH)
        def _(e0):
            chunk(base + e0, *A, *B, True, True)
            chunk(base + e0 + CH, *B, *A, True, True)

        chunk(base + (NCH - 2) * CH, *A, *B, True, True)
        chunk(base + (NCH - 1) * CH, *B, *A, True, False)

        pltpu.async_copy(src_hbm.at[pl.ds(base + NCH * CH, TAIL)], src_t, isem)
        pltpu.async_copy(dst_hbm.at[pl.ds(base + NCH * CH, TAIL)], dst_t, isem)
        pltpu.make_async_copy(src_hbm.at[pl.ds(base + NCH * CH, TAIL)], src_t, isem).wait()
        pltpu.make_async_copy(dst_hbm.at[pl.ds(base + NCH * CH, TAIL)], dst_t, isem).wait()
        pltpu.async_copy(g_hbm.at[src_t], rows_t, gsem1).wait()
        pltpu.sync_copy(rows_t, acc.at[dst_t], add=True)

        pltpu.make_async_copy(rows_b, acc.at[dst_b], ssem_b).wait()
        plsc.subcore_barrier()

        for r in range(0, RPT, ZR):
            pltpu.async_copy(acc.at[pl.ds(s * RPT + r, ZR)],
                             out_hbm.at[c, pl.ds(s * RPT + r, ZR)], zsem)
        for r in range(0, RPT, ZR):
            pltpu.make_async_copy(acc.at[pl.ds(s * RPT + r, ZR)],
                                  out_hbm.at[c, pl.ds(s * RPT + r, ZR)], zsem).wait()

    return hop_kernel(g, src, dst, zeros128)


def _norm_from_deg(d_ref):
    deg = d_ref[0] + d_ref[1]
    return lax.rsqrt(jnp.maximum(deg, 1.0))[:, 0:1]


def _tc_scale(x, degp):
    def body(x_ref, d_ref, o_ref):
        o_ref[...] = x_ref[...] * _norm_from_deg(d_ref)

    return pl.pallas_call(
        body,
        grid=(N // BN,),
        in_specs=[
            pl.BlockSpec((BN, D), lambda i: (i, 0)),
            pl.BlockSpec((NC, BN, D), lambda i: (0, i, 0)),
        ],
        out_specs=pl.BlockSpec((BN, D), lambda i: (i, 0)),
        out_shape=jax.ShapeDtypeStruct((N, D), jnp.float32),
    )(x, degp)


def _tc_mid(aggp, degp):
    def body(a_ref, d_ref, h_ref, g_ref):
        nrm = _norm_from_deg(d_ref)
        h = (a_ref[0] + a_ref[1]) * nrm
        h_ref[...] = h
        g_ref[...] = h * nrm

    return pl.pallas_call(
        body,
        grid=(N // BN,),
        in_specs=[
            pl.BlockSpec((NC, BN, D), lambda i: (0, i, 0)),
            pl.BlockSpec((NC, BN, D), lambda i: (0, i, 0)),
        ],
        out_specs=[
            pl.BlockSpec((BN, D), lambda i: (i, 0)),
            pl.BlockSpec((BN, D), lambda i: (i, 0)),
        ],
        out_shape=[
            jax.ShapeDtypeStruct((N, D), jnp.float32),
            jax.ShapeDtypeStruct((N, D), jnp.float32),
        ],
    )(aggp, degp)


def _tc_layer1_end(f0, h1, aggp2, degp, Wt, b):
    def body(f0_ref, h1_ref, a_ref, d_ref, w_ref, b_ref, y_ref, g_ref):
        nrm = _norm_from_deg(d_ref)
        h2 = (a_ref[0] + a_ref[1]) * nrm
        cat = jnp.concatenate([f0_ref[...], h1_ref[...], h2], axis=1)
        y = jnp.dot(cat, w_ref[...], preferred_element_type=jnp.float32)
        y = jnp.maximum(y + b_ref[...], 0.0)
        y_ref[...] = y
        g_ref[...] = y * nrm

    return pl.pallas_call(
        body,
        grid=(N // BN,),
        in_specs=[
            pl.BlockSpec((BN, D), lambda i: (i, 0)),
            pl.BlockSpec((BN, D), lambda i: (i, 0)),
            pl.BlockSpec((NC, BN, D), lambda i: (0, i, 0)),
            pl.BlockSpec((NC, BN, D), lambda i: (0, i, 0)),
            pl.BlockSpec((3 * D, D), lambda i: (0, 0)),
            pl.BlockSpec((1, D), lambda i: (0, 0)),
        ],
        out_specs=[
            pl.BlockSpec((BN, D), lambda i: (i, 0)),
            pl.BlockSpec((BN, D), lambda i: (i, 0)),
        ],
        out_shape=[
            jax.ShapeDtypeStruct((N, D), jnp.float32),
            jax.ShapeDtypeStruct((N, D), jnp.float32),
        ],
    )(f0, h1, aggp2, degp, Wt, b)


def _tc_layer2_end(f0, h1, aggp2, degp, Wt, b):
    def body(f0_ref, h1_ref, a_ref, d_ref, w_ref, b_ref, o_ref):
        i = pl.program_id(0)

        @pl.when(i == 0)
        def _():
            o_ref[...] = jnp.zeros_like(o_ref)

        nrm = _norm_from_deg(d_ref)
        h2 = (a_ref[0] + a_ref[1]) * nrm
        cat = jnp.concatenate([f0_ref[...], h1_ref[...], h2], axis=1)
        y = jnp.dot(cat, w_ref[...], preferred_element_type=jnp.float32)
        y = jnp.maximum(y + b_ref[...], 0.0)
        o_ref[...] = jnp.maximum(o_ref[...], jnp.max(y, axis=0, keepdims=True))

    return pl.pallas_call(
        body,
        grid=(N // BN,),
        in_specs=[
            pl.BlockSpec((BN, D), lambda i: (i, 0)),
            pl.BlockSpec((BN, D), lambda i: (i, 0)),
            pl.BlockSpec((NC, BN, D), lambda i: (0, i, 0)),
            pl.BlockSpec((NC, BN, D), lambda i: (0, i, 0)),
            pl.BlockSpec((3 * D, D), lambda i: (0, 0)),
            pl.BlockSpec((1, D), lambda i: (0, 0)),
        ],
        out_specs=pl.BlockSpec((1, D), lambda i: (0, 0)),
        out_shape=jax.ShapeDtypeStruct((1, D), jnp.float32),
    )(f0, h1, aggp2, degp, Wt, b)


def kernel(x, edge_index, W1, b1, W2, b2):
    src = edge_index[0]
    dst = edge_index[1]
    zeros128 = jnp.zeros((ZR, D), jnp.float32)
    ones128 = jnp.ones((CH, D), jnp.float32)
    W1t = W1.T
    W2t = W2.T
    b1r = b1.reshape(1, D)
    b2r = b2.reshape(1, D)

    degp = _sc_degree(dst, zeros128, ones128)

    g0 = _tc_scale(x, degp)
    a1 = _sc_hop(g0, src, dst, zeros128)
    h1, g1 = _tc_mid(a1, degp)
    a2 = _sc_hop(g1, src, dst, zeros128)
    y1, gy1 = _tc_layer1_end(x, h1, a2, degp, W1t, b1r)

    a3 = _sc_hop(gy1, src, dst, zeros128)
    h1b, g1b = _tc_mid(a3, degp)
    a4 = _sc_hop(g1b, src, dst, zeros128)
    return _tc_layer2_end(y1, h1b, a4, degp, W2t, b2r)
